# initial kernel scaffold (unmeasured)
import jax
import jax.numpy as jnp
from jax import lax
from jax.experimental import pallas as pl
from jax.experimental.pallas import tpu as pltpu

N_DEV = 4
M_PER = 1024
K = 4096
N = 8192
N_PER = 2048
NBLK = 512
GRID = N // NBLK


def _gemm_body(x_ref, w_ref, y_ref, amax_ref, acc_ref):
    j = pl.program_id(0)

    @pl.when(j == 0)
    def _():
        acc_ref[...] = jnp.zeros_like(acc_ref)

    yblk = jnp.dot(x_ref[...], w_ref[...], preferred_element_type=jnp.float32)
    y_ref[...] = yblk
    acc_ref[...] = jnp.maximum(acc_ref[...], jnp.max(jnp.abs(yblk)))

    @pl.when(j == GRID - 1)
    def _():
        amax_ref[...] = acc_ref[...]


def _gemm(x, w):
    return pl.pallas_call(
        _gemm_body,
        grid=(GRID,),
        in_specs=[
            pl.BlockSpec((M_PER, K), lambda j: (0, 0)),
            pl.BlockSpec((K, NBLK), lambda j: (0, j)),
        ],
        out_specs=[
            pl.BlockSpec((M_PER, NBLK), lambda j: (0, j)),
            pl.BlockSpec((8, 128), lambda j: (0, 0)),
        ],
        out_shape=[
            jax.ShapeDtypeStruct((M_PER, N), jnp.float32),
            jax.ShapeDtypeStruct((8, 128), jnp.float32),
        ],
        scratch_shapes=[pltpu.VMEM((8, 128), jnp.float32)],
    )(x, w)


def _a2a_body(y_ref, amax_ref, out_ref, amax_all, dsend, drecv, asend,
              arecv, local_sem):
    me = lax.axis_index("i")

    barrier = pltpu.get_barrier_semaphore()
    for off in (1, 2, 3):
        pl.semaphore_signal(
            barrier, inc=1,
            device_id=((me + off) % N_DEV,),
            device_id_type=pl.DeviceIdType.MESH,
        )
    pl.semaphore_wait(barrier, 3)

    local_cp = pltpu.make_async_copy(
        y_ref.at[:, pl.ds(me * N_PER, N_PER)],
        out_ref.at[pl.ds(me * M_PER, M_PER), :],
        local_sem,
    )
    local_cp.start()

    amax_all[pl.ds(me, 1), :, :] = amax_ref[...].reshape(1, 8, 128)

    sends = []
    for off in (1, 2, 3):
        p = (me + off) % N_DEV
        inv = N_DEV - off
        a = pltpu.make_async_remote_copy(
            src_ref=amax_all.at[pl.ds(me, 1)],
            dst_ref=amax_all.at[pl.ds(me, 1)],
            send_sem=asend.at[off],
            recv_sem=arecv.at[inv],
            device_id=(p,),
            device_id_type=pl.DeviceIdType.MESH,
        )
        a.start()
        d = pltpu.make_async_remote_copy(
            src_ref=y_ref.at[:, pl.ds(p * N_PER, N_PER)],
            dst_ref=out_ref.at[pl.ds(me * M_PER, M_PER), :],
            send_sem=dsend.at[off],
            recv_sem=drecv.at[inv],
            device_id=(p,),
            device_id_type=pl.DeviceIdType.MESH,
        )
        d.start()
        sends.append(a)
        sends.append(d)

    for off in (1, 2, 3):
        s = (me + off) % N_DEV
        recv_a = pltpu.make_async_remote_copy(
            src_ref=amax_all.at[pl.ds(me, 1)],
            dst_ref=amax_all.at[pl.ds(s, 1)],
            send_sem=asend.at[0],
            recv_sem=arecv.at[off],
            device_id=(s,),
            device_id_type=pl.DeviceIdType.MESH,
        )
        recv_a.wait_recv()
        recv_d = pltpu.make_async_remote_copy(
            src_ref=y_ref.at[:, pl.ds(0, N_PER)],
            dst_ref=out_ref.at[pl.ds(s * M_PER, M_PER), :],
            send_sem=dsend.at[0],
            recv_sem=drecv.at[off],
            device_id=(s,),
            device_id_type=pl.DeviceIdType.MESH,
        )
        recv_d.wait_recv()

    local_cp.wait()

    amax = jnp.max(amax_all[...])
    scale = amax / 127.0
    q = jnp.clip(jnp.round(out_ref[...] * (127.0 / amax)), -127.0, 127.0)
    out_ref[...] = q * scale

    for s in sends:
        s.wait_send()


def _a2a(y, amax):
    return pl.pallas_call(
        _a2a_body,
        in_specs=[
            pl.BlockSpec(memory_space=pltpu.ANY),
            pl.BlockSpec(memory_space=pltpu.VMEM),
        ],
        out_specs=pl.BlockSpec(memory_space=pltpu.VMEM),
        out_shape=jax.ShapeDtypeStruct((N_DEV * M_PER, N_PER), jnp.float32),
        scratch_shapes=[
            pltpu.VMEM((N_DEV, 8, 128), jnp.float32),
            pltpu.SemaphoreType.DMA((N_DEV,)),
            pltpu.SemaphoreType.DMA((N_DEV,)),
            pltpu.SemaphoreType.DMA((N_DEV,)),
            pltpu.SemaphoreType.DMA((N_DEV,)),
            pltpu.SemaphoreType.DMA,
        ],
        compiler_params=pltpu.CompilerParams(collective_id=0),
    )(y, amax)


def kernel(x, w_mat):
    y, amax = _gemm(x, w_mat)
    return _a2a(y, amax)


# baseline (device time: 316498 ns/iter reference)
import jax
import jax.numpy as jnp
from jax import lax
from jax.experimental import pallas as pl
from jax.experimental.pallas import tpu as pltpu

N_DEV = 4
M_PER = 1024
K = 4096
N = 8192
N_PER = 2048
NBLK = 512
GRID = N // NBLK


def _gemm_body(x_ref, w_ref, y_ref, amax_ref, acc_ref):
    j = pl.program_id(0)

    @pl.when(j == 0)
    def _():
        acc_ref[...] = jnp.zeros_like(acc_ref)

    yblk = jnp.dot(x_ref[...], w_ref[...], preferred_element_type=jnp.float32)
    y_ref[...] = yblk
    acc_ref[...] = jnp.maximum(acc_ref[...], jnp.max(jnp.abs(yblk)))

    @pl.when(j == GRID - 1)
    def _():
        amax_ref[...] = acc_ref[...]


def _gemm(x, w):
    return pl.pallas_call(
        _gemm_body,
        grid=(GRID,),
        in_specs=[
            pl.BlockSpec((M_PER, K), lambda j: (0, 0)),
            pl.BlockSpec((K, NBLK), lambda j: (0, j)),
        ],
        out_specs=[
            pl.BlockSpec((M_PER, NBLK), lambda j: (0, j)),
            pl.BlockSpec((8, 128), lambda j: (0, 0)),
        ],
        out_shape=[
            jax.ShapeDtypeStruct((M_PER, N), jnp.float32),
            jax.ShapeDtypeStruct((8, 128), jnp.float32),
        ],
        scratch_shapes=[pltpu.VMEM((8, 128), jnp.float32)],
        compiler_params=pltpu.CompilerParams(
            vmem_limit_bytes=100 * 1024 * 1024,
        ),
    )(x, w)


def _a2a_body(y_ref, amax_ref, out_ref, amax_all, dsend, drecv, asend,
              arecv, local_sem):
    me = lax.axis_index("i")

    barrier = pltpu.get_barrier_semaphore()
    for off in (1, 2, 3):
        pl.semaphore_signal(
            barrier, inc=1,
            device_id=((me + off) % N_DEV,),
            device_id_type=pl.DeviceIdType.MESH,
        )
    pl.semaphore_wait(barrier, 3)

    local_cp = pltpu.make_async_copy(
        y_ref.at[:, pl.ds(me * N_PER, N_PER)],
        out_ref.at[pl.ds(me * M_PER, M_PER), :],
        local_sem,
    )
    local_cp.start()

    amax_all[pl.ds(me, 1), :, :] = amax_ref[...].reshape(1, 8, 128)

    sends = []
    for off in (1, 2, 3):
        p = (me + off) % N_DEV
        inv = N_DEV - off
        a = pltpu.make_async_remote_copy(
            src_ref=amax_all.at[pl.ds(me, 1)],
            dst_ref=amax_all.at[pl.ds(me, 1)],
            send_sem=asend.at[off],
            recv_sem=arecv.at[inv],
            device_id=(p,),
            device_id_type=pl.DeviceIdType.MESH,
        )
        a.start()
        d = pltpu.make_async_remote_copy(
            src_ref=y_ref.at[:, pl.ds(p * N_PER, N_PER)],
            dst_ref=out_ref.at[pl.ds(me * M_PER, M_PER), :],
            send_sem=dsend.at[off],
            recv_sem=drecv.at[inv],
            device_id=(p,),
            device_id_type=pl.DeviceIdType.MESH,
        )
        d.start()
        sends.append(a)
        sends.append(d)

    for off in (1, 2, 3):
        s = (me + off) % N_DEV
        recv_a = pltpu.make_async_remote_copy(
            src_ref=amax_all.at[pl.ds(me, 1)],
            dst_ref=amax_all.at[pl.ds(s, 1)],
            send_sem=asend.at[0],
            recv_sem=arecv.at[off],
            device_id=(s,),
            device_id_type=pl.DeviceIdType.MESH,
        )
        recv_a.wait_recv()
        recv_d = pltpu.make_async_remote_copy(
            src_ref=y_ref.at[:, pl.ds(0, N_PER)],
            dst_ref=out_ref.at[pl.ds(s * M_PER, M_PER), :],
            send_sem=dsend.at[0],
            recv_sem=drecv.at[off],
            device_id=(s,),
            device_id_type=pl.DeviceIdType.MESH,
        )
        recv_d.wait_recv()

    local_cp.wait()

    amax = jnp.max(amax_all[...])
    scale = amax / 127.0
    q = jnp.clip(jnp.round(out_ref[...] * (127.0 / amax)), -127.0, 127.0)
    out_ref[...] = q * scale

    for s in sends:
        s.wait_send()


def _a2a(y, amax):
    return pl.pallas_call(
        _a2a_body,
        in_specs=[
            pl.BlockSpec(memory_space=pl.ANY),
            pl.BlockSpec(memory_space=pltpu.VMEM),
        ],
        out_specs=pl.BlockSpec(memory_space=pltpu.VMEM),
        out_shape=jax.ShapeDtypeStruct((N_DEV * M_PER, N_PER), jnp.float32),
        scratch_shapes=[
            pltpu.VMEM((N_DEV, 8, 128), jnp.float32),
            pltpu.SemaphoreType.DMA((N_DEV,)),
            pltpu.SemaphoreType.DMA((N_DEV,)),
            pltpu.SemaphoreType.DMA((N_DEV,)),
            pltpu.SemaphoreType.DMA((N_DEV,)),
            pltpu.SemaphoreType.DMA,
        ],
        compiler_params=pltpu.CompilerParams(
            collective_id=0,
            vmem_limit_bytes=100 * 1024 * 1024,
        ),
    )(y, amax)


def kernel(x, w_mat):
    y, amax = _gemm(x, w_mat)
    return _a2a(y, amax)


# device time: 178815 ns/iter; 1.7700x vs baseline; 1.7700x over previous
import jax
import jax.numpy as jnp
from jax import lax
from jax.experimental import pallas as pl
from jax.experimental.pallas import tpu as pltpu

N_DEV = 4
M_PER = 1024
K = 4096
N = 8192
N_PER = 2048
NBLK = 512
GRID = N // NBLK


def _gemm_body(x_ref, w_ref, y_ref, amax_ref, acc_ref):
    j = pl.program_id(0)

    @pl.when(j == 0)
    def _():
        acc_ref[...] = jnp.zeros_like(acc_ref)

    yblk = jnp.dot(x_ref[...], w_ref[...], preferred_element_type=jnp.float32)
    y_ref[...] = yblk
    acc_ref[...] = jnp.maximum(acc_ref[...], jnp.max(jnp.abs(yblk)))

    @pl.when(j == GRID - 1)
    def _():
        amax_ref[...] = acc_ref[...]


def _gemm(x, w):
    return pl.pallas_call(
        _gemm_body,
        grid=(GRID,),
        in_specs=[
            pl.BlockSpec((M_PER, K), lambda j: (0, 0)),
            pl.BlockSpec((K, NBLK), lambda j: (0, j)),
        ],
        out_specs=[
            pl.BlockSpec((M_PER, NBLK), lambda j: (0, j)),
            pl.BlockSpec((8, 128), lambda j: (0, 0)),
        ],
        out_shape=[
            jax.ShapeDtypeStruct((M_PER, N), jnp.float32),
            jax.ShapeDtypeStruct((8, 128), jnp.float32),
        ],
        scratch_shapes=[pltpu.VMEM((8, 128), jnp.float32)],
        compiler_params=pltpu.CompilerParams(
            vmem_limit_bytes=100 * 1024 * 1024,
        ),
    )(x, w)


def _a2a_body(y_ref, amax_ref, out_ref, amax_all, ystage, qsend, qrecv,
              fstage, dsend, drecv, asend, arecv, ysem, osem):
    me = lax.axis_index("i")

    barrier = pltpu.get_barrier_semaphore()
    for off in (1, 2, 3):
        pl.semaphore_signal(
            barrier, inc=1,
            device_id=((me + off) % N_DEV,),
            device_id_type=pl.DeviceIdType.MESH,
        )
    pl.semaphore_wait(barrier, 3)

    def chunk_dma(off, buf):
        p = (me + off) % N_DEV
        return pltpu.make_async_copy(
            y_ref.at[:, pl.ds(p * N_PER, N_PER)],
            ystage.at[buf],
            ysem.at[buf],
        )

    offs = (1, 2, 3, 0)
    pend_y = {0: chunk_dma(offs[0], 0)}
    pend_y[0].start()

    amax_all[pl.ds(me, 1), :, :] = amax_ref[...].reshape(1, 8, 128)
    amax_sends = []
    for off in (1, 2, 3):
        p = (me + off) % N_DEV
        inv = N_DEV - off
        a = pltpu.make_async_remote_copy(
            src_ref=amax_all.at[pl.ds(me, 1)],
            dst_ref=amax_all.at[pl.ds(me, 1)],
            send_sem=asend.at[off],
            recv_sem=arecv.at[inv],
            device_id=(p,),
            device_id_type=pl.DeviceIdType.MESH,
        )
        a.start()
        amax_sends.append(a)
    for off in (1, 2, 3):
        s = (me + off) % N_DEV
        recv_a = pltpu.make_async_remote_copy(
            src_ref=amax_all.at[pl.ds(me, 1)],
            dst_ref=amax_all.at[pl.ds(s, 1)],
            send_sem=asend.at[0],
            recv_sem=arecv.at[off],
            device_id=(s,),
            device_id_type=pl.DeviceIdType.MESH,
        )
        recv_a.wait_recv()

    gamax = jnp.max(amax_all[...])
    inv_s = 127.0 / gamax
    scale = gamax / 127.0

    data_sends = []
    out_cps = {}
    for idx, off in enumerate(offs):
        buf = idx % 2
        if idx + 1 < len(offs):
            nxt = chunk_dma(offs[idx + 1], (idx + 1) % 2)
            nxt.start()
            pend_y[(idx + 1) % 2] = nxt
        pend_y[buf].wait()
        q = jnp.clip(jnp.round(ystage[buf] * inv_s), -127.0, 127.0)
        if off != 0:
            qsend[off] = q.astype(jnp.int8)
            p = (me + off) % N_DEV
            inv = N_DEV - off
            d = pltpu.make_async_remote_copy(
                src_ref=qsend.at[off],
                dst_ref=qrecv.at[inv],
                send_sem=dsend.at[off],
                recv_sem=drecv.at[inv],
                device_id=(p,),
                device_id_type=pl.DeviceIdType.MESH,
            )
            d.start()
            data_sends.append(d)
        else:
            fstage[0] = q * scale
            ocp = pltpu.make_async_copy(
                fstage.at[0],
                out_ref.at[pl.ds(me * M_PER, M_PER), :],
                osem.at[0],
            )
            ocp.start()
            out_cps[0] = ocp

    fslots = (1, 0, 1)
    for k, off in enumerate((1, 2, 3)):
        s = (me + off) % N_DEV
        recv_d = pltpu.make_async_remote_copy(
            src_ref=qsend.at[0],
            dst_ref=qrecv.at[off],
            send_sem=dsend.at[0],
            recv_sem=drecv.at[off],
            device_id=(s,),
            device_id_type=pl.DeviceIdType.MESH,
        )
        recv_d.wait_recv()
        fs = fslots[k]
        if fs in out_cps:
            out_cps[fs].wait()
        fstage[fs] = qrecv[off].astype(jnp.float32) * scale
        ocp = pltpu.make_async_copy(
            fstage.at[fs],
            out_ref.at[pl.ds(s * M_PER, M_PER), :],
            osem.at[fs],
        )
        ocp.start()
        out_cps[fs] = ocp

    out_cps[0].wait()
    out_cps[1].wait()
    for d in data_sends:
        d.wait_send()
    for a in amax_sends:
        a.wait_send()


def _a2a(y, amax):
    return pl.pallas_call(
        _a2a_body,
        in_specs=[
            pl.BlockSpec(memory_space=pl.ANY),
            pl.BlockSpec(memory_space=pltpu.VMEM),
        ],
        out_specs=pl.BlockSpec(memory_space=pl.ANY),
        out_shape=jax.ShapeDtypeStruct((N_DEV * M_PER, N_PER), jnp.float32),
        scratch_shapes=[
            pltpu.VMEM((N_DEV, 8, 128), jnp.float32),
            pltpu.VMEM((2, M_PER, N_PER), jnp.float32),
            pltpu.VMEM((N_DEV, M_PER, N_PER), jnp.int8),
            pltpu.VMEM((N_DEV, M_PER, N_PER), jnp.int8),
            pltpu.VMEM((2, M_PER, N_PER), jnp.float32),
            pltpu.SemaphoreType.DMA((N_DEV,)),
            pltpu.SemaphoreType.DMA((N_DEV,)),
            pltpu.SemaphoreType.DMA((N_DEV,)),
            pltpu.SemaphoreType.DMA((N_DEV,)),
            pltpu.SemaphoreType.DMA((2,)),
            pltpu.SemaphoreType.DMA((2,)),
        ],
        compiler_params=pltpu.CompilerParams(
            collective_id=0,
            vmem_limit_bytes=100 * 1024 * 1024,
        ),
    )(y, amax)


def kernel(x, w_mat):
    y, amax = _gemm(x, w_mat)
    return _a2a(y, amax)


# device time: 175798 ns/iter; 1.8004x vs baseline; 1.0172x over previous
import jax
import jax.numpy as jnp
from jax import lax
from jax.experimental import pallas as pl
from jax.experimental.pallas import tpu as pltpu

N_DEV = 4
M_PER = 1024
K = 4096
N = 8192
N_PER = 2048
H_ROWS = 512
NBLK = 1024
GRID = N // NBLK


def _gemm_body(x_ref, w_ref, y_ref, amax_ref, acc_ref):
    j = pl.program_id(0)

    @pl.when(j == 0)
    def _():
        acc_ref[...] = jnp.zeros_like(acc_ref)

    yblk = jnp.dot(x_ref[...], w_ref[...], preferred_element_type=jnp.float32)
    y_ref[...] = yblk
    acc_ref[...] = jnp.maximum(
        acc_ref[...],
        jnp.max(jnp.abs(yblk).reshape(128, 8, NBLK), axis=0),
    )

    @pl.when(j == GRID - 1)
    def _():
        amax_ref[...] = jnp.full((8, 128), jnp.max(acc_ref[...]),
                                 dtype=jnp.float32)


def _gemm(x, w):
    return pl.pallas_call(
        _gemm_body,
        grid=(GRID,),
        in_specs=[
            pl.BlockSpec((M_PER, K), lambda j: (0, 0)),
            pl.BlockSpec((K, NBLK), lambda j: (0, j)),
        ],
        out_specs=[
            pl.BlockSpec((M_PER, NBLK), lambda j: (0, j)),
            pl.BlockSpec((8, 128), lambda j: (0, 0)),
        ],
        out_shape=[
            jax.ShapeDtypeStruct((M_PER, N), jnp.float32),
            jax.ShapeDtypeStruct((8, 128), jnp.float32),
        ],
        scratch_shapes=[pltpu.VMEM((8, NBLK), jnp.float32)],
        compiler_params=pltpu.CompilerParams(
            vmem_limit_bytes=100 * 1024 * 1024,
        ),
    )(x, w)


_A2A_OFFS = (1, 2, 3)


def _a2a_body(y_ref, amax_ref, out_ref, amax_all, ystage, qsend, qrecv,
              fstage, dsend, drecv, asend, arecv, ysem, osem):
    me = lax.axis_index("i")

    barrier = pltpu.get_barrier_semaphore()
    for off in (1, 2, 3):
        pl.semaphore_signal(
            barrier, inc=1,
            device_id=((me + off) % N_DEV,),
            device_id_type=pl.DeviceIdType.MESH,
        )
    pl.semaphore_wait(barrier, 3)

    def sub_dma(off, h, buf):
        p = (me + off) % N_DEV
        return pltpu.make_async_copy(
            y_ref.at[pl.ds(h * H_ROWS, H_ROWS), pl.ds(p * N_PER, N_PER)],
            ystage.at[buf],
            ysem.at[buf],
        )

    seq = tuple((off, h) for off in _A2A_OFFS + (0,) for h in (0, 1))
    pend_y = {0: sub_dma(seq[0][0], seq[0][1], 0)}
    pend_y[0].start()

    amax_all[pl.ds(me, 1), :, :] = amax_ref[...].reshape(1, 8, 128)
    amax_sends = []
    for off in (1, 2, 3):
        p = (me + off) % N_DEV
        inv = N_DEV - off
        a = pltpu.make_async_remote_copy(
            src_ref=amax_all.at[pl.ds(me, 1)],
            dst_ref=amax_all.at[pl.ds(me, 1)],
            send_sem=asend.at[off],
            recv_sem=arecv.at[inv],
            device_id=(p,),
            device_id_type=pl.DeviceIdType.MESH,
        )
        a.start()
        amax_sends.append(a)
    for off in (1, 2, 3):
        s = (me + off) % N_DEV
        recv_a = pltpu.make_async_remote_copy(
            src_ref=amax_all.at[pl.ds(me, 1)],
            dst_ref=amax_all.at[pl.ds(s, 1)],
            send_sem=asend.at[0],
            recv_sem=arecv.at[off],
            device_id=(s,),
            device_id_type=pl.DeviceIdType.MESH,
        )
        recv_a.wait_recv()

    gamax = jnp.max(amax_all[...])
    inv_s = 127.0 / gamax
    scale = gamax / 127.0

    data_sends = []
    out_cps = {}
    fs_next = 0
    for idx, (off, h) in enumerate(seq):
        buf = idx % 2
        if idx + 1 < len(seq):
            noff, nh = seq[idx + 1]
            nxt = sub_dma(noff, nh, (idx + 1) % 2)
            nxt.start()
            pend_y[(idx + 1) % 2] = nxt
        pend_y[buf].wait()
        q = jnp.clip(jnp.round(ystage[buf] * inv_s), -127.0, 127.0)
        if off != 0:
            slot = off * 2 + h
            qsend[slot] = q.astype(jnp.int8)
            p = (me + off) % N_DEV
            inv = N_DEV - off
            d = pltpu.make_async_remote_copy(
                src_ref=qsend.at[slot],
                dst_ref=qrecv.at[inv * 2 + h],
                send_sem=dsend.at[slot],
                recv_sem=drecv.at[inv * 2 + h],
                device_id=(p,),
                device_id_type=pl.DeviceIdType.MESH,
            )
            d.start()
            data_sends.append(d)
        else:
            fs = fs_next
            fs_next = 1 - fs_next
            if fs in out_cps:
                out_cps[fs].wait()
            fstage[fs] = q * scale
            ocp = pltpu.make_async_copy(
                fstage.at[fs],
                out_ref.at[pl.ds(me * M_PER + h * H_ROWS, H_ROWS), :],
                osem.at[fs],
            )
            ocp.start()
            out_cps[fs] = ocp

    for off in _A2A_OFFS:
        s = (me + off) % N_DEV
        for h in (0, 1):
            slot = off * 2 + h
            recv_d = pltpu.make_async_remote_copy(
                src_ref=qsend.at[0],
                dst_ref=qrecv.at[slot],
                send_sem=dsend.at[0],
                recv_sem=drecv.at[slot],
                device_id=(s,),
                device_id_type=pl.DeviceIdType.MESH,
            )
            recv_d.wait_recv()
            fs = fs_next
            fs_next = 1 - fs_next
            if fs in out_cps:
                out_cps[fs].wait()
            fstage[fs] = qrecv[slot].astype(jnp.float32) * scale
            ocp = pltpu.make_async_copy(
                fstage.at[fs],
                out_ref.at[pl.ds(s * M_PER + h * H_ROWS, H_ROWS), :],
                osem.at[fs],
            )
            ocp.start()
            out_cps[fs] = ocp

    out_cps[0].wait()
    out_cps[1].wait()
    for d in data_sends:
        d.wait_send()
    for a in amax_sends:
        a.wait_send()


def _a2a(y, amax):
    return pl.pallas_call(
        _a2a_body,
        in_specs=[
            pl.BlockSpec(memory_space=pl.ANY),
            pl.BlockSpec(memory_space=pltpu.VMEM),
        ],
        out_specs=pl.BlockSpec(memory_space=pl.ANY),
        out_shape=jax.ShapeDtypeStruct((N_DEV * M_PER, N_PER), jnp.float32),
        scratch_shapes=[
            pltpu.VMEM((N_DEV, 8, 128), jnp.float32),
            pltpu.VMEM((2, H_ROWS, N_PER), jnp.float32),
            pltpu.VMEM((2 * N_DEV, H_ROWS, N_PER), jnp.int8),
            pltpu.VMEM((2 * N_DEV, H_ROWS, N_PER), jnp.int8),
            pltpu.VMEM((2, H_ROWS, N_PER), jnp.float32),
            pltpu.SemaphoreType.DMA((2 * N_DEV,)),
            pltpu.SemaphoreType.DMA((2 * N_DEV,)),
            pltpu.SemaphoreType.DMA((N_DEV,)),
            pltpu.SemaphoreType.DMA((N_DEV,)),
            pltpu.SemaphoreType.DMA((2,)),
            pltpu.SemaphoreType.DMA((2,)),
        ],
        compiler_params=pltpu.CompilerParams(
            collective_id=0,
            vmem_limit_bytes=100 * 1024 * 1024,
        ),
    )(y, amax)


def kernel(x, w_mat):
    y, amax = _gemm(x, w_mat)
    return _a2a(y, amax)


# device time: 154855 ns/iter; 2.0438x vs baseline; 1.1352x over previous
import jax
import jax.numpy as jnp
from jax import lax
from jax.experimental import pallas as pl
from jax.experimental.pallas import tpu as pltpu

N_DEV = 4
M_PER = 1024
K = 4096
N = 8192
N_PER = 2048
H_COLS = 1024
WBLK = 512
STEPS = N // WBLK


def _gemm_send_body(x_hbm, w_hbm, y_hbm, amax_ref, bfrecv_hbm,
                    xbuf, wbuf, ybuf, bfstage, acc_ref,
                    xsem, wsem, yosem, bfsend, brecv):
    me = lax.axis_index("i")

    barrier = pltpu.get_barrier_semaphore()
    for off in (1, 2, 3):
        pl.semaphore_signal(
            barrier, inc=1,
            device_id=((me + off) % N_DEV,),
            device_id_type=pl.DeviceIdType.MESH,
        )
    pl.semaphore_wait(barrier, 3)

    xcp = pltpu.make_async_copy(x_hbm, xbuf, xsem)
    xcp.start()

    def colbase(s):
        t, c = s // 4, s % 4
        dest = (me + 1 + t) % N_DEV
        return dest * N_PER + c * WBLK

    def w_dma(s):
        return pltpu.make_async_copy(
            w_hbm.at[:, pl.ds(colbase(s), WBLK)], wbuf.at[s % 2],
            wsem.at[s % 2],
        )

    w_dma(0).start()
    acc_ref[...] = jnp.zeros_like(acc_ref)
    xcp.wait()

    bf_sends = {}
    y_cps = {}
    for s in range(STEPS):
        t, c = s // 4, s % 4
        dest = (me + 1 + t) % N_DEV
        if s + 1 < STEPS:
            w_dma(s + 1).start()
        pltpu.make_async_copy(
            w_hbm.at[:, pl.ds(colbase(s), WBLK)], wbuf.at[s % 2],
            wsem.at[s % 2],
        ).wait()
        yb = jnp.dot(xbuf[...], wbuf[s % 2],
                     preferred_element_type=jnp.float32)
        slot = s % 2
        if slot in y_cps:
            y_cps[slot].wait()
        ybuf[slot] = yb
        ycp = pltpu.make_async_copy(
            ybuf.at[slot], y_hbm.at[:, pl.ds(colbase(s), WBLK)],
            yosem.at[slot],
        )
        ycp.start()
        y_cps[slot] = ycp
        acc_ref[...] = jnp.maximum(
            acc_ref[...],
            jnp.max(jnp.abs(yb).reshape(128, 8, WBLK), axis=0),
        )
        if t < 3 and c < 2:
            if c == 0 and t >= 2:
                bf_sends[t - 2].wait_send()
            bfstage[pl.ds(t % 2, 1), :, pl.ds(c * WBLK, WBLK)] = (
                yb.astype(jnp.bfloat16).reshape(1, M_PER, WBLK))
            if c == 1:
                o_recv = 3 - t
                d = pltpu.make_async_remote_copy(
                    src_ref=bfstage.at[t % 2],
                    dst_ref=bfrecv_hbm.at[o_recv],
                    send_sem=bfsend.at[t],
                    recv_sem=brecv.at[o_recv],
                    device_id=(dest,),
                    device_id_type=pl.DeviceIdType.MESH,
                )
                d.start()
                bf_sends[t] = d

    amax_ref[...] = jnp.full((8, 128), jnp.max(acc_ref[...]),
                             dtype=jnp.float32)

    y_cps[0].wait()
    y_cps[1].wait()
    for t in (1, 2):
        bf_sends[t].wait_send()
    for o in (1, 2, 3):
        s_dev = (me + o) % N_DEV
        pltpu.make_async_remote_copy(
            src_ref=bfstage.at[0],
            dst_ref=bfrecv_hbm.at[o],
            send_sem=bfsend.at[0],
            recv_sem=brecv.at[o],
            device_id=(s_dev,),
            device_id_type=pl.DeviceIdType.MESH,
        ).wait_recv()


def _gemm_send(x, w):
    return pl.pallas_call(
        _gemm_send_body,
        in_specs=[
            pl.BlockSpec(memory_space=pl.ANY),
            pl.BlockSpec(memory_space=pl.ANY),
        ],
        out_specs=[
            pl.BlockSpec(memory_space=pl.ANY),
            pl.BlockSpec(memory_space=pltpu.VMEM),
            pl.BlockSpec(memory_space=pl.ANY),
        ],
        out_shape=[
            jax.ShapeDtypeStruct((M_PER, N), jnp.float32),
            jax.ShapeDtypeStruct((8, 128), jnp.float32),
            jax.ShapeDtypeStruct((N_DEV, M_PER, H_COLS), jnp.bfloat16),
        ],
        scratch_shapes=[
            pltpu.VMEM((M_PER, K), jnp.float32),
            pltpu.VMEM((2, K, WBLK), jnp.float32),
            pltpu.VMEM((2, M_PER, WBLK), jnp.float32),
            pltpu.VMEM((2, M_PER, H_COLS), jnp.bfloat16),
            pltpu.VMEM((8, WBLK), jnp.float32),
            pltpu.SemaphoreType.DMA,
            pltpu.SemaphoreType.DMA((2,)),
            pltpu.SemaphoreType.DMA((2,)),
            pltpu.SemaphoreType.DMA((N_DEV,)),
            pltpu.SemaphoreType.DMA((N_DEV,)),
        ],
        compiler_params=pltpu.CompilerParams(
            collective_id=1,
            vmem_limit_bytes=100 * 1024 * 1024,
        ),
    )(x, w)


def _a2a_body(y_ref, amax_ref, bfrecv_ref, out_ref, amax_all, ystage,
              qsend, qrecv, bstage, fstage, dsend, drecv, asend, arecv,
              ysem, bsem, osem):
    me = lax.axis_index("i")

    barrier = pltpu.get_barrier_semaphore()
    for off in (1, 2, 3):
        pl.semaphore_signal(
            barrier, inc=1,
            device_id=((me + off) % N_DEV,),
            device_id_type=pl.DeviceIdType.MESH,
        )
    pl.semaphore_wait(barrier, 3)

    def half_dma(off, h, buf):
        p = (me + off) % N_DEV
        return pltpu.make_async_copy(
            y_ref.at[:, pl.ds(p * N_PER + h * H_COLS, H_COLS)],
            ystage.at[buf],
            ysem.at[buf],
        )

    seq = ((1, 1), (2, 1), (3, 1), (0, 0), (0, 1))
    pend_y = {0: half_dma(*seq[0], 0)}
    pend_y[0].start()

    amax_all[pl.ds(me, 1), :, :] = amax_ref[...].reshape(1, 8, 128)
    amax_sends = []
    for off in (1, 2, 3):
        p = (me + off) % N_DEV
        inv = N_DEV - off
        a = pltpu.make_async_remote_copy(
            src_ref=amax_all.at[pl.ds(me, 1)],
            dst_ref=amax_all.at[pl.ds(me, 1)],
            send_sem=asend.at[off],
            recv_sem=arecv.at[inv],
            device_id=(p,),
            device_id_type=pl.DeviceIdType.MESH,
        )
        a.start()
        amax_sends.append(a)
    for off in (1, 2, 3):
        s = (me + off) % N_DEV
        pltpu.make_async_remote_copy(
            src_ref=amax_all.at[pl.ds(me, 1)],
            dst_ref=amax_all.at[pl.ds(s, 1)],
            send_sem=asend.at[0],
            recv_sem=arecv.at[off],
            device_id=(s,),
            device_id_type=pl.DeviceIdType.MESH,
        ).wait_recv()

    gamax = jnp.max(amax_all[...])
    inv_s = 127.0 / gamax
    scale = gamax / 127.0

    out_cps = {}
    fs_next = 0

    def stage_out(val, rows, cols):
        nonlocal fs_next
        fs = fs_next
        fs_next = 1 - fs_next
        if fs in out_cps:
            out_cps[fs].wait()
        fstage[fs] = val
        ocp = pltpu.make_async_copy(
            fstage.at[fs],
            out_ref.at[pl.ds(rows, M_PER), pl.ds(cols, H_COLS)],
            osem.at[fs],
        )
        ocp.start()
        out_cps[fs] = ocp

    data_sends = []
    for idx, (off, h) in enumerate(seq):
        buf = idx % 2
        if idx + 1 < len(seq):
            nxt = half_dma(*seq[idx + 1], (idx + 1) % 2)
            nxt.start()
            pend_y[(idx + 1) % 2] = nxt
        pend_y[buf].wait()
        q = jnp.clip(jnp.round(ystage[buf] * inv_s), -127.0, 127.0)
        if off != 0:
            qsend[off] = q.astype(jnp.int8)
            p = (me + off) % N_DEV
            inv = N_DEV - off
            d = pltpu.make_async_remote_copy(
                src_ref=qsend.at[off],
                dst_ref=qrecv.at[inv],
                send_sem=dsend.at[off],
                recv_sem=drecv.at[inv],
                device_id=(p,),
                device_id_type=pl.DeviceIdType.MESH,
            )
            d.start()
            data_sends.append(d)
        else:
            stage_out(q * scale, me * M_PER, h * H_COLS)

    bf_cps = {}
    for o in (1, 2, 3):
        bcp = pltpu.make_async_copy(bfrecv_ref.at[o], bstage.at[o - 1],
                                    bsem.at[o - 1])
        bcp.start()
        bf_cps[o] = bcp
    for o in (1, 2, 3):
        s = (me + o) % N_DEV
        bf_cps[o].wait()
        val = bstage[o - 1].astype(jnp.float32)
        q = jnp.clip(jnp.round(val * inv_s), -127.0, 127.0)
        stage_out(q * scale, s * M_PER, 0)

    for off in (1, 2, 3):
        s = (me + off) % N_DEV
        pltpu.make_async_remote_copy(
            src_ref=qsend.at[0],
            dst_ref=qrecv.at[off],
            send_sem=dsend.at[0],
            recv_sem=drecv.at[off],
            device_id=(s,),
            device_id_type=pl.DeviceIdType.MESH,
        ).wait_recv()
        stage_out(qrecv[off].astype(jnp.float32) * scale,
                  s * M_PER, H_COLS)

    out_cps[0].wait()
    out_cps[1].wait()
    for d in data_sends:
        d.wait_send()
    for a in amax_sends:
        a.wait_send()


def _a2a(y, amax, bfrecv):
    return pl.pallas_call(
        _a2a_body,
        in_specs=[
            pl.BlockSpec(memory_space=pl.ANY),
            pl.BlockSpec(memory_space=pltpu.VMEM),
            pl.BlockSpec(memory_space=pl.ANY),
        ],
        out_specs=pl.BlockSpec(memory_space=pl.ANY),
        out_shape=jax.ShapeDtypeStruct((N_DEV * M_PER, N_PER), jnp.float32),
        scratch_shapes=[
            pltpu.VMEM((N_DEV, 8, 128), jnp.float32),
            pltpu.VMEM((2, M_PER, H_COLS), jnp.float32),
            pltpu.VMEM((N_DEV, M_PER, H_COLS), jnp.int8),
            pltpu.VMEM((N_DEV, M_PER, H_COLS), jnp.int8),
            pltpu.VMEM((3, M_PER, H_COLS), jnp.bfloat16),
            pltpu.VMEM((2, M_PER, H_COLS), jnp.float32),
            pltpu.SemaphoreType.DMA((N_DEV,)),
            pltpu.SemaphoreType.DMA((N_DEV,)),
            pltpu.SemaphoreType.DMA((N_DEV,)),
            pltpu.SemaphoreType.DMA((N_DEV,)),
            pltpu.SemaphoreType.DMA((2,)),
            pltpu.SemaphoreType.DMA((3,)),
            pltpu.SemaphoreType.DMA((2,)),
        ],
        compiler_params=pltpu.CompilerParams(
            collective_id=0,
            vmem_limit_bytes=100 * 1024 * 1024,
        ),
    )(y, amax, bfrecv)


def kernel(x, w_mat):
    y, amax, bfrecv = _gemm_send(x, w_mat)
    return _a2a(y, amax, bfrecv)
